# baseline (speedup 1.0000x reference)
import functools

import jax
import jax.numpy as jnp
from jax import lax
from jax.experimental import pallas as pl
from jax.experimental.pallas import tpu as pltpu
from jax.experimental.pallas import tpu_sc as plsc

N = 100000
E = 1600000
D = 128
H = 4
NB = 32
SLOPE = 0.2

BS = 512
NP = 100352
NSC = 2
NTILE = 16
ROWS_PER_TILE = NP // NTILE

C2 = 512
NCHT2 = E // C2
NW2 = NSC * NTILE
NJ2 = (NCHT2 + NW2 - 1) // NW2
C3 = 640
NCHT3 = E // C3
NJ3 = (NCHT3 + NTILE - 1) // NTILE


def _leaky(v):
    return jnp.where(v >= 0, v, SLOPE * v)


def _stage1_body(x_ref, wcat_ref, as_ref, ad_ref, h_ref, ss_ref, sd_ref):
    h = jnp.dot(x_ref[...], wcat_ref[...], preferred_element_type=jnp.float32)
    h_ref[...] = h
    ss_ref[...] = jnp.dot(h, as_ref[...].T, preferred_element_type=jnp.float32)
    sd_ref[...] = jnp.dot(h, ad_ref[...].T, preferred_element_type=jnp.float32)


def _stage1(x_pad, Wcat, As16, Ad16):
    return pl.pallas_call(
        _stage1_body,
        out_shape=(
            jax.ShapeDtypeStruct((NP, H * D), jnp.float32),
            jax.ShapeDtypeStruct((NP, 16), jnp.float32),
            jax.ShapeDtypeStruct((NP, 16), jnp.float32),
        ),
        grid=(NP // BS,),
        in_specs=[
            pl.BlockSpec((BS, D), lambda i: (i, 0)),
            pl.BlockSpec((D, H * D), lambda i: (0, 0)),
            pl.BlockSpec((16, H * D), lambda i: (0, 0)),
            pl.BlockSpec((16, H * D), lambda i: (0, 0)),
        ],
        out_specs=(
            pl.BlockSpec((BS, H * D), lambda i: (i, 0)),
            pl.BlockSpec((BS, 16), lambda i: (i, 0)),
            pl.BlockSpec((BS, 16), lambda i: (i, 0)),
        ),
    )(x_pad, Wcat, As16, Ad16)


def _stage2_body(ei_hbm, ss_hbm, sd_hbm, z16_hbm,
                 wt_hbm, den_hbm,
                 src_v, dst_v, srows_v, drows_v, w2_v, wt_v, den_sh,
                 sem, sem2):
    c = lax.axis_index("c")
    s = lax.axis_index("s")
    wid = s * NSC + c

    pltpu.sync_copy(z16_hbm,
                    den_sh.at[pl.ds(s * ROWS_PER_TILE, ROWS_PER_TILE)])
    plsc.subcore_barrier()

    lanes = lax.iota(jnp.int32, 16)

    def chunk(j, _):
        jj = j * NW2 + wid

        @pl.when(jj < NCHT2)
        def _():
            base = jj * C2
            pltpu.sync_copy(ei_hbm.at[0].at[pl.ds(base, C2)], src_v)
            pltpu.sync_copy(ei_hbm.at[1].at[pl.ds(base, C2)], dst_v)
            cp1 = pltpu.make_async_copy(ss_hbm.at[src_v], srows_v, sem)
            cp2 = pltpu.make_async_copy(sd_hbm.at[dst_v], drows_v, sem2)
            cp1.start()
            cp2.start()
            cp1.wait()
            cp2.wait()

            def grp(i, _):
                ridx = 16 * i + lanes
                r = ridx >> 2
                q = ridx & 3
                a = plsc.load_gather(srows_v, [r, q])
                bq = plsc.load_gather(drows_v, [r, q])
                w = jnp.exp(_leaky(a + bq))
                plsc.store_scatter(w2_v, [r, q], w)
                return 0
            lax.fori_loop(0, C2 * H // 16, grp, 0)

            pltpu.sync_copy(w2_v, den_sh.at[dst_v], add=True)

            def tgrp(i, _):
                r = 16 * i + lanes
                for k in range(H):
                    col = plsc.load_gather(
                        w2_v, [r, jnp.full((16,), k, jnp.int32)])
                    wt_v[pl.ds(k * C2 + 16 * i, 16)] = col
                return 0
            lax.fori_loop(0, C2 // 16, tgrp, 0)
            for k in range(H):
                pltpu.sync_copy(wt_v.at[pl.ds(k * C2, C2)],
                                wt_hbm.at[k].at[pl.ds(base, C2)])
        return 0

    lax.fori_loop(0, NJ2, chunk, 0)

    plsc.subcore_barrier()
    pltpu.sync_copy(den_sh.at[pl.ds(s * ROWS_PER_TILE, ROWS_PER_TILE)],
                    den_hbm.at[c].at[pl.ds(s * ROWS_PER_TILE, ROWS_PER_TILE)])


def _stage2(edge_index, ss, sd, z16):
    mesh = plsc.VectorSubcoreMesh(core_axis_name="c", subcore_axis_name="s")
    f = pl.kernel(
        _stage2_body,
        out_type=(
            jax.ShapeDtypeStruct((H, E), jnp.float32),
            jax.ShapeDtypeStruct((NSC, NP, 16), jnp.float32),
        ),
        mesh=mesh,
        scratch_types=[
            pltpu.VMEM((C2,), jnp.int32),
            pltpu.VMEM((C2,), jnp.int32),
            pltpu.VMEM((C2, 16), jnp.float32),
            pltpu.VMEM((C2, 16), jnp.float32),
            pltpu.VMEM((C2, 16), jnp.float32),
            pltpu.VMEM((H * C2,), jnp.float32),
            pltpu.VMEM_SHARED((NP, 16), jnp.float32),
            pltpu.SemaphoreType.DMA,
            pltpu.SemaphoreType.DMA,
        ],
        compiler_params=pltpu.CompilerParams(needs_layout_passes=False, use_tc_tiling_on_sc=False),
    )
    return f(edge_index, ss, sd, z16)


def _stage3_body(ei_hbm, hflat_hbm, wt_hbm, z16_hbm,
                 acc_hbm,
                 srcA, dstA, idxA, wA, dscA, rowsA,
                 srcB, dstB, idxB, wB, dscB, rowsB,
                 acc_sh, semlA, semlB, semgA, semgB, semsA, semsB):
    c = lax.axis_index("c")
    s = lax.axis_index("s")
    setA = (srcA, dstA, idxA, wA, dscA, rowsA, semlA, semgA, semsA)
    setB = (srcB, dstB, idxB, wB, dscB, rowsB, semlB, semgB, semsB)
    nvalid = (NCHT3 - 1 - s) // NTILE + 1

    def rnd(r, _):
        b = r * NSC + c
        k = b >> 3

        pltpu.sync_copy(z16_hbm,
                        acc_sh.at[pl.ds(s * ROWS_PER_TILE, ROWS_PER_TILE)])
        plsc.subcore_barrier()

        def when_valid(x, f):
            if isinstance(x, int) and x < 0:
                return

            def g():
                f()

            pl.when((x >= 0) & (x < nvalid))(g)

        def base_of(x):
            return (x * NTILE + s) * C3

        def lin_copies(x, st):
            base = base_of(x)
            return (
                pltpu.make_async_copy(ei_hbm.at[0].at[pl.ds(base, C3)],
                                      st[0], st[6]),
                pltpu.make_async_copy(ei_hbm.at[1].at[pl.ds(base, C3)],
                                      st[1], st[6]),
                pltpu.make_async_copy(wt_hbm.at[k].at[pl.ds(base, C3)],
                                      st[3], st[6]),
            )

        def start_lin(x, st):
            when_valid(x, lambda: [cp.start() for cp in lin_copies(x, st)])

        def wait_lin(x, st):
            when_valid(x, lambda: [cp.wait() for cp in lin_copies(x, st)])

        def gat_copy(st):
            return pltpu.make_async_copy(hflat_hbm.at[st[2]], st[5], st[7])

        def sca_copy(st):
            return pltpu.make_async_copy(st[5], acc_sh.at[st[4]], st[8])

        def sca_start(st):
            pltpu.async_copy(st[5], acc_sh.at[st[4]], st[8], add=True)

        def idx_compute(st):
            def gidx(i, _):
                sl = pl.ds(16 * i, 16)
                st[2][sl] = st[0][sl] * NB + b
                return 0
            lax.fori_loop(0, C3 // 16, gidx, 0)

        def scale_and_scatter(st):
            def scale(g, _):
                for u in range(2):
                    gg = 2 * g + u
                    sl = pl.ds(16 * gg, 16)
                    st[4][sl] = st[1][sl]
                    wv = st[3][sl]
                    for t in range(16):
                        e = 16 * gg + t
                        rr = st[5].at[e]
                        rr[...] = rr[...] * wv[t]
                return 0
            lax.fori_loop(0, C3 // 32, scale, 0)
            sca_start(st)

        start_lin(0, setA)

        def pair(jp, _):
            j0 = 2 * jp
            j1 = 2 * jp + 1
            when_valid(j0, lambda: (wait_lin(j0, setA), idx_compute(setA)))
            when_valid(j0 - 2, lambda: sca_copy(setA).wait())
            when_valid(j0, lambda: gat_copy(setA).start())
            when_valid(j0 - 1, lambda: (gat_copy(setB).wait(),
                                        scale_and_scatter(setB)))
            start_lin(j1, setB)
            when_valid(j1, lambda: (wait_lin(j1, setB), idx_compute(setB)))
            when_valid(j1 - 2, lambda: sca_copy(setB).wait())
            when_valid(j1, lambda: gat_copy(setB).start())
            when_valid(j0, lambda: (gat_copy(setA).wait(),
                                    scale_and_scatter(setA)))
            start_lin(j0 + 2, setA)
            return 0

        npair = (NJ3 + 1) // 2
        lax.fori_loop(0, npair, pair, 0)
        when_valid(2 * npair - 2, lambda: sca_copy(setA).wait())
        when_valid(2 * npair - 1, lambda: sca_copy(setB).wait())

        plsc.subcore_barrier()
        pltpu.sync_copy(acc_sh.at[pl.ds(s * ROWS_PER_TILE, ROWS_PER_TILE)],
                        acc_hbm.at[b].at[pl.ds(s * ROWS_PER_TILE, ROWS_PER_TILE)])
        plsc.subcore_barrier()
        return 0

    lax.fori_loop(0, NB // NSC, rnd, 0)


def _stage3(edge_index, hflat, wt, z16):
    mesh = plsc.VectorSubcoreMesh(core_axis_name="c", subcore_axis_name="s")
    bufset = [
        pltpu.VMEM((C3,), jnp.int32),
        pltpu.VMEM((C3,), jnp.int32),
        pltpu.VMEM((C3,), jnp.int32),
        pltpu.VMEM((C3,), jnp.float32),
        pltpu.VMEM((C3,), jnp.int32),
        pltpu.VMEM((C3, 16), jnp.float32),
    ]
    f = pl.kernel(
        _stage3_body,
        out_type=jax.ShapeDtypeStruct((NB, NP, 16), jnp.float32),
        mesh=mesh,
        scratch_types=bufset + bufset + [
            pltpu.VMEM_SHARED((NP, 16), jnp.float32),
        ] + [pltpu.SemaphoreType.DMA] * 6,
        compiler_params=pltpu.CompilerParams(needs_layout_passes=False, use_tc_tiling_on_sc=False),
    )
    return f(edge_index, hflat, wt, z16)


def _stage4_body(acc_ref, den_ref, h_ref, ss_ref, sd_ref, bcat_ref,
                 outw_ref, outb_ref, o_ref):
    wself = jnp.exp(_leaky(ss_ref[...] + sd_ref[...]))[:, :H]
    dent = (den_ref[0] + den_ref[1])[:, :H] + wself
    hk = h_ref[...].reshape(BS, H, D)
    acck = acc_ref[...].reshape(BS, H, D)
    outc = (acck + wself[:, :, None] * hk) / dent[:, :, None]
    cat = outc.reshape(BS, H * D) + bcat_ref[...]
    o = jnp.dot(cat, outw_ref[...], preferred_element_type=jnp.float32)
    o_ref[...] = jnp.maximum(o + outb_ref[...], 0.0)


def _stage4(acc, den, h_cat, ss, sd, bcat, outW, outb):
    return pl.pallas_call(
        _stage4_body,
        out_shape=jax.ShapeDtypeStruct((NP, D), jnp.float32),
        grid=(NP // BS,),
        in_specs=[
            pl.BlockSpec((BS, H * D), lambda i: (i, 0)),
            pl.BlockSpec((NSC, BS, 16), lambda i: (0, i, 0)),
            pl.BlockSpec((BS, H * D), lambda i: (i, 0)),
            pl.BlockSpec((BS, 16), lambda i: (i, 0)),
            pl.BlockSpec((BS, 16), lambda i: (i, 0)),
            pl.BlockSpec((1, H * D), lambda i: (0, 0)),
            pl.BlockSpec((H * D, D), lambda i: (0, 0)),
            pl.BlockSpec((1, D), lambda i: (0, 0)),
        ],
        out_specs=pl.BlockSpec((BS, D), lambda i: (i, 0)),
    )(acc, den, h_cat, ss, sd, bcat, outW, outb)


def _layer(x_pad, edge_index, heads, outW, outb, z16):
    Wcat = jnp.concatenate([h[0] for h in heads], axis=1)
    As16 = jnp.zeros((16, H * D), jnp.float32)
    Ad16 = jnp.zeros((16, H * D), jnp.float32)
    for k in range(H):
        As16 = As16.at[k, k * D:(k + 1) * D].set(heads[k][1])
        Ad16 = Ad16.at[k, k * D:(k + 1) * D].set(heads[k][2])
    bcat = jnp.concatenate([h[3] for h in heads])[None]

    h_cat, ss, sd = _stage1(x_pad, Wcat, As16, Ad16)
    wt, den = _stage2(edge_index, ss, sd, z16)
    hflat = h_cat.reshape(NP * NB, 16)
    acc = _stage3(edge_index, hflat, wt, z16)
    accv = jnp.transpose(acc, (1, 0, 2)).reshape(NP, H * D)
    return _stage4(accv, den, h_cat, ss, sd, bcat, outW, outb[None])


def kernel(x, edge_index, enc_W0, enc_as0, enc_ad0, enc_b0, enc_W1, enc_as1, enc_ad1, enc_b1, enc_W2, enc_as2, enc_ad2, enc_b2, enc_W3, enc_as3, enc_ad3, enc_b3, enc_outW, enc_outb, dec_W0, dec_as0, dec_ad0, dec_b0, dec_W1, dec_as1, dec_ad1, dec_b1, dec_W2, dec_as2, dec_ad2, dec_b2, dec_W3, dec_as3, dec_ad3, dec_b3, dec_outW, dec_outb):
    enc_heads = [(enc_W0, enc_as0, enc_ad0, enc_b0),
                 (enc_W1, enc_as1, enc_ad1, enc_b1),
                 (enc_W2, enc_as2, enc_ad2, enc_b2),
                 (enc_W3, enc_as3, enc_ad3, enc_b3)]
    dec_heads = [(dec_W0, dec_as0, dec_ad0, dec_b0),
                 (dec_W1, dec_as1, dec_ad1, dec_b1),
                 (dec_W2, dec_as2, dec_ad2, dec_b2),
                 (dec_W3, dec_as3, dec_ad3, dec_b3)]

    z16 = jnp.zeros((ROWS_PER_TILE, 16), jnp.float32)

    x_pad = jnp.pad(x, ((0, NP - N), (0, 0)))
    enc = _layer(x_pad, edge_index, enc_heads, enc_outW, enc_outb, z16)
    dec = _layer(enc, edge_index, dec_heads, dec_outW, dec_outb, z16)
    return dec[:N]


# speedup vs baseline: 1.0182x; 1.0182x over previous
import functools

import jax
import jax.numpy as jnp
from jax import lax
from jax.experimental import pallas as pl
from jax.experimental.pallas import tpu as pltpu
from jax.experimental.pallas import tpu_sc as plsc

N = 100000
E = 1600000
D = 128
H = 4
NB = 32
SLOPE = 0.2

BS = 512
NP = 100352
NSC = 2
NTILE = 16
ROWS_PER_TILE = NP // NTILE

C2 = 512
NCHT2 = E // C2
NW2 = NSC * NTILE
NJ2 = (NCHT2 + NW2 - 1) // NW2
C3 = 640
NCHT3 = E // C3
NJ3 = (NCHT3 + NTILE - 1) // NTILE


def _leaky(v):
    return jnp.where(v >= 0, v, SLOPE * v)


def _stage1_body(x_ref, wcat_ref, as_ref, ad_ref, h_ref, ss_ref, sd_ref):
    h = jnp.dot(x_ref[...], wcat_ref[...], preferred_element_type=jnp.float32)
    h_ref[...] = h
    ss_ref[...] = jnp.dot(h, as_ref[...].T, preferred_element_type=jnp.float32)
    sd_ref[...] = jnp.dot(h, ad_ref[...].T, preferred_element_type=jnp.float32)


def _stage1(x_pad, Wcat, As16, Ad16):
    return pl.pallas_call(
        _stage1_body,
        out_shape=(
            jax.ShapeDtypeStruct((NP, H * D), jnp.float32),
            jax.ShapeDtypeStruct((NP, 16), jnp.float32),
            jax.ShapeDtypeStruct((NP, 16), jnp.float32),
        ),
        grid=(NP // BS,),
        in_specs=[
            pl.BlockSpec((BS, D), lambda i: (i, 0)),
            pl.BlockSpec((D, H * D), lambda i: (0, 0)),
            pl.BlockSpec((16, H * D), lambda i: (0, 0)),
            pl.BlockSpec((16, H * D), lambda i: (0, 0)),
        ],
        out_specs=(
            pl.BlockSpec((BS, H * D), lambda i: (i, 0)),
            pl.BlockSpec((BS, 16), lambda i: (i, 0)),
            pl.BlockSpec((BS, 16), lambda i: (i, 0)),
        ),
    )(x_pad, Wcat, As16, Ad16)


def _stage2_body(ei_hbm, ss_hbm, sd_hbm, z16_hbm,
                 wt_hbm, den_hbm,
                 src_v, dst_v, srows_v, drows_v, w2_v, wt_v, den_sh,
                 sem, sem2):
    c = lax.axis_index("c")
    s = lax.axis_index("s")
    wid = s * NSC + c

    pltpu.sync_copy(z16_hbm,
                    den_sh.at[pl.ds(s * ROWS_PER_TILE, ROWS_PER_TILE)])
    plsc.subcore_barrier()

    lanes = lax.iota(jnp.int32, 16)

    def chunk(j, _):
        jj = j * NW2 + wid

        @pl.when(jj < NCHT2)
        def _():
            base = jj * C2
            pltpu.sync_copy(ei_hbm.at[0].at[pl.ds(base, C2)], src_v)
            pltpu.sync_copy(ei_hbm.at[1].at[pl.ds(base, C2)], dst_v)
            cp1 = pltpu.make_async_copy(ss_hbm.at[src_v], srows_v, sem)
            cp2 = pltpu.make_async_copy(sd_hbm.at[dst_v], drows_v, sem2)
            cp1.start()
            cp2.start()
            cp1.wait()
            cp2.wait()

            def rows(g, _):
                for t in range(4):
                    e = 4 * g + t
                    w = jnp.exp(_leaky(srows_v[e] + drows_v[e]))
                    w2_v.at[e][...] = w
                return 0
            lax.fori_loop(0, C2 // 4, rows, 0)

            pltpu.sync_copy(w2_v, den_sh.at[dst_v], add=True)

            def tgrp(i, _):
                r = 16 * i + lanes
                for k in range(H):
                    col = plsc.load_gather(
                        w2_v, [r, jnp.full((16,), k, jnp.int32)])
                    wt_v[pl.ds(k * C2 + 16 * i, 16)] = col
                return 0
            lax.fori_loop(0, C2 // 16, tgrp, 0)
            for k in range(H):
                pltpu.sync_copy(wt_v.at[pl.ds(k * C2, C2)],
                                wt_hbm.at[k].at[pl.ds(base, C2)])
        return 0

    lax.fori_loop(0, NJ2, chunk, 0)

    plsc.subcore_barrier()
    pltpu.sync_copy(den_sh.at[pl.ds(s * ROWS_PER_TILE, ROWS_PER_TILE)],
                    den_hbm.at[c].at[pl.ds(s * ROWS_PER_TILE, ROWS_PER_TILE)])


def _stage2(edge_index, ss, sd, z16):
    mesh = plsc.VectorSubcoreMesh(core_axis_name="c", subcore_axis_name="s")
    f = pl.kernel(
        _stage2_body,
        out_type=(
            jax.ShapeDtypeStruct((H, E), jnp.float32),
            jax.ShapeDtypeStruct((NSC, NP, 16), jnp.float32),
        ),
        mesh=mesh,
        scratch_types=[
            pltpu.VMEM((C2,), jnp.int32),
            pltpu.VMEM((C2,), jnp.int32),
            pltpu.VMEM((C2, 16), jnp.float32),
            pltpu.VMEM((C2, 16), jnp.float32),
            pltpu.VMEM((C2, 16), jnp.float32),
            pltpu.VMEM((H * C2,), jnp.float32),
            pltpu.VMEM_SHARED((NP, 16), jnp.float32),
            pltpu.SemaphoreType.DMA,
            pltpu.SemaphoreType.DMA,
        ],
        compiler_params=pltpu.CompilerParams(needs_layout_passes=False, use_tc_tiling_on_sc=False),
    )
    return f(edge_index, ss, sd, z16)


def _stage3_body(ei_hbm, hflat_hbm, wt_hbm, z16_hbm,
                 acc_hbm,
                 srcA, dstA, idxA, wA, dscA, rowsA,
                 srcB, dstB, idxB, wB, dscB, rowsB,
                 acc_sh, semlA, semlB, semgA, semgB, semsA, semsB):
    c = lax.axis_index("c")
    s = lax.axis_index("s")
    setA = (srcA, dstA, idxA, wA, dscA, rowsA, semlA, semgA, semsA)
    setB = (srcB, dstB, idxB, wB, dscB, rowsB, semlB, semgB, semsB)
    nvalid = (NCHT3 - 1 - s) // NTILE + 1

    def rnd(r, _):
        b = r * NSC + c
        k = b >> 3

        pltpu.sync_copy(z16_hbm,
                        acc_sh.at[pl.ds(s * ROWS_PER_TILE, ROWS_PER_TILE)])
        plsc.subcore_barrier()

        def when_valid(x, f):
            if isinstance(x, int) and x < 0:
                return

            def g():
                f()

            pl.when((x >= 0) & (x < nvalid))(g)

        def base_of(x):
            return (x * NTILE + s) * C3

        def lin_copies(x, st):
            base = base_of(x)
            return (
                pltpu.make_async_copy(ei_hbm.at[0].at[pl.ds(base, C3)],
                                      st[0], st[6]),
                pltpu.make_async_copy(ei_hbm.at[1].at[pl.ds(base, C3)],
                                      st[1], st[6]),
                pltpu.make_async_copy(wt_hbm.at[k].at[pl.ds(base, C3)],
                                      st[3], st[6]),
            )

        def start_lin(x, st):
            when_valid(x, lambda: [cp.start() for cp in lin_copies(x, st)])

        def wait_lin(x, st):
            when_valid(x, lambda: [cp.wait() for cp in lin_copies(x, st)])

        def gat_copy(st):
            return pltpu.make_async_copy(hflat_hbm.at[st[2]], st[5], st[7])

        def sca_copy(st):
            return pltpu.make_async_copy(st[5], acc_sh.at[st[4]], st[8])

        def sca_start(st):
            pltpu.async_copy(st[5], acc_sh.at[st[4]], st[8], add=True)

        def idx_compute(st):
            def gidx(i, _):
                sl = pl.ds(16 * i, 16)
                st[2][sl] = st[0][sl] * NB + b
                return 0
            lax.fori_loop(0, C3 // 16, gidx, 0)

        def scale_and_scatter(st):
            def scale(g, _):
                for u in range(2):
                    gg = 2 * g + u
                    sl = pl.ds(16 * gg, 16)
                    st[4][sl] = st[1][sl]
                    wv = st[3][sl]
                    for t in range(16):
                        e = 16 * gg + t
                        rr = st[5].at[e]
                        rr[...] = rr[...] * wv[t]
                return 0
            lax.fori_loop(0, C3 // 32, scale, 0)
            sca_start(st)

        start_lin(0, setA)

        def pair(jp, _):
            j0 = 2 * jp
            j1 = 2 * jp + 1
            when_valid(j0, lambda: (wait_lin(j0, setA), idx_compute(setA)))
            when_valid(j0 - 2, lambda: sca_copy(setA).wait())
            when_valid(j0, lambda: gat_copy(setA).start())
            when_valid(j0 - 1, lambda: (gat_copy(setB).wait(),
                                        scale_and_scatter(setB)))
            start_lin(j1, setB)
            when_valid(j1, lambda: (wait_lin(j1, setB), idx_compute(setB)))
            when_valid(j1 - 2, lambda: sca_copy(setB).wait())
            when_valid(j1, lambda: gat_copy(setB).start())
            when_valid(j0, lambda: (gat_copy(setA).wait(),
                                    scale_and_scatter(setA)))
            start_lin(j0 + 2, setA)
            return 0

        npair = (NJ3 + 1) // 2
        lax.fori_loop(0, npair, pair, 0)
        when_valid(2 * npair - 2, lambda: sca_copy(setA).wait())
        when_valid(2 * npair - 1, lambda: sca_copy(setB).wait())

        plsc.subcore_barrier()
        pltpu.sync_copy(acc_sh.at[pl.ds(s * ROWS_PER_TILE, ROWS_PER_TILE)],
                        acc_hbm.at[b].at[pl.ds(s * ROWS_PER_TILE, ROWS_PER_TILE)])
        plsc.subcore_barrier()
        return 0

    lax.fori_loop(0, NB // NSC, rnd, 0)


def _stage3(edge_index, hflat, wt, z16):
    mesh = plsc.VectorSubcoreMesh(core_axis_name="c", subcore_axis_name="s")
    bufset = [
        pltpu.VMEM((C3,), jnp.int32),
        pltpu.VMEM((C3,), jnp.int32),
        pltpu.VMEM((C3,), jnp.int32),
        pltpu.VMEM((C3,), jnp.float32),
        pltpu.VMEM((C3,), jnp.int32),
        pltpu.VMEM((C3, 16), jnp.float32),
    ]
    f = pl.kernel(
        _stage3_body,
        out_type=jax.ShapeDtypeStruct((NB, NP, 16), jnp.float32),
        mesh=mesh,
        scratch_types=bufset + bufset + [
            pltpu.VMEM_SHARED((NP, 16), jnp.float32),
        ] + [pltpu.SemaphoreType.DMA] * 6,
        compiler_params=pltpu.CompilerParams(needs_layout_passes=False, use_tc_tiling_on_sc=False),
    )
    return f(edge_index, hflat, wt, z16)


def _stage4_body(acc_ref, den_ref, h_ref, ss_ref, sd_ref, bcat_ref,
                 outw_ref, outb_ref, o_ref):
    wself = jnp.exp(_leaky(ss_ref[...] + sd_ref[...]))[:, :H]
    dent = (den_ref[0] + den_ref[1])[:, :H] + wself
    hk = h_ref[...].reshape(BS, H, D)
    acck = acc_ref[...].reshape(BS, H, D)
    outc = (acck + wself[:, :, None] * hk) / dent[:, :, None]
    cat = outc.reshape(BS, H * D) + bcat_ref[...]
    o = jnp.dot(cat, outw_ref[...], preferred_element_type=jnp.float32)
    o_ref[...] = jnp.maximum(o + outb_ref[...], 0.0)


def _stage4(acc, den, h_cat, ss, sd, bcat, outW, outb):
    return pl.pallas_call(
        _stage4_body,
        out_shape=jax.ShapeDtypeStruct((NP, D), jnp.float32),
        grid=(NP // BS,),
        in_specs=[
            pl.BlockSpec((BS, H * D), lambda i: (i, 0)),
            pl.BlockSpec((NSC, BS, 16), lambda i: (0, i, 0)),
            pl.BlockSpec((BS, H * D), lambda i: (i, 0)),
            pl.BlockSpec((BS, 16), lambda i: (i, 0)),
            pl.BlockSpec((BS, 16), lambda i: (i, 0)),
            pl.BlockSpec((1, H * D), lambda i: (0, 0)),
            pl.BlockSpec((H * D, D), lambda i: (0, 0)),
            pl.BlockSpec((1, D), lambda i: (0, 0)),
        ],
        out_specs=pl.BlockSpec((BS, D), lambda i: (i, 0)),
    )(acc, den, h_cat, ss, sd, bcat, outW, outb)


def _layer(x_pad, edge_index, heads, outW, outb, z16):
    Wcat = jnp.concatenate([h[0] for h in heads], axis=1)
    As16 = jnp.zeros((16, H * D), jnp.float32)
    Ad16 = jnp.zeros((16, H * D), jnp.float32)
    for k in range(H):
        As16 = As16.at[k, k * D:(k + 1) * D].set(heads[k][1])
        Ad16 = Ad16.at[k, k * D:(k + 1) * D].set(heads[k][2])
    bcat = jnp.concatenate([h[3] for h in heads])[None]

    h_cat, ss, sd = _stage1(x_pad, Wcat, As16, Ad16)
    wt, den = _stage2(edge_index, ss, sd, z16)
    hflat = h_cat.reshape(NP * NB, 16)
    acc = _stage3(edge_index, hflat, wt, z16)
    accv = jnp.transpose(acc, (1, 0, 2)).reshape(NP, H * D)
    return _stage4(accv, den, h_cat, ss, sd, bcat, outW, outb[None])


def kernel(x, edge_index, enc_W0, enc_as0, enc_ad0, enc_b0, enc_W1, enc_as1, enc_ad1, enc_b1, enc_W2, enc_as2, enc_ad2, enc_b2, enc_W3, enc_as3, enc_ad3, enc_b3, enc_outW, enc_outb, dec_W0, dec_as0, dec_ad0, dec_b0, dec_W1, dec_as1, dec_ad1, dec_b1, dec_W2, dec_as2, dec_ad2, dec_b2, dec_W3, dec_as3, dec_ad3, dec_b3, dec_outW, dec_outb):
    enc_heads = [(enc_W0, enc_as0, enc_ad0, enc_b0),
                 (enc_W1, enc_as1, enc_ad1, enc_b1),
                 (enc_W2, enc_as2, enc_ad2, enc_b2),
                 (enc_W3, enc_as3, enc_ad3, enc_b3)]
    dec_heads = [(dec_W0, dec_as0, dec_ad0, dec_b0),
                 (dec_W1, dec_as1, dec_ad1, dec_b1),
                 (dec_W2, dec_as2, dec_ad2, dec_b2),
                 (dec_W3, dec_as3, dec_ad3, dec_b3)]

    z16 = jnp.zeros((ROWS_PER_TILE, 16), jnp.float32)

    x_pad = jnp.pad(x, ((0, NP - N), (0, 0)))
    enc = _layer(x_pad, edge_index, enc_heads, enc_outW, enc_outb, z16)
    dec = _layer(enc, edge_index, dec_heads, dec_outW, dec_outb, z16)
    return dec[:N]
